# baseline (speedup 1.0000x reference)
import functools

import jax
import jax.numpy as jnp
from jax import lax
from jax.experimental import pallas as pl
from jax.experimental.pallas import tpu as pltpu
from jax.experimental.pallas import tpu_sc as plsc

N = 10000
D = 128
E = 320000

NC = 2
NS = 16
NW = NC * NS
C = 128
M = 80
SG = 40
NSTG = M // SG
EPAD = NW * M * C
NACC = 10112
RT = NACC // NS
NDUM = NACC - N

_mesh = plsc.VectorSubcoreMesh(core_axis_name="c", subcore_axis_name="s")

DW = 16


def _fill(ref, val):
    def body(i, carry):
        for t in range(D // 16):
            ref[i, pl.ds(16 * t, 16)] = jnp.full((16,), val, jnp.float32)
        return carry

    lax.fori_loop(0, ref.shape[0], body, 0)


def _zero_acc(zbuf, acc_sh, r0):
    for k in range(4):
        pltpu.sync_copy(zbuf, acc_sh.at[pl.ds(r0 + 128 * k, 128)])
    pltpu.sync_copy(zbuf.at[pl.ds(0, RT - 512)],
                    acc_sh.at[pl.ds(r0 + 512, RT - 512)])


@functools.partial(
    pl.kernel,
    out_type=jax.ShapeDtypeStruct((NC, NACC, D), jnp.float32),
    mesh=_mesh,
    scratch_types=[
        pltpu.VMEM((M, C), jnp.int32),
        pltpu.VMEM((C, D), jnp.float32),
        pltpu.VMEM_SHARED((NACC, D), jnp.float32),
        pltpu.SemaphoreType.DMA,
    ],
)
def _sc_deg(dstb, degp, idx_d, ones_v, acc_sh, dsem):
    c = lax.axis_index("c")
    s = lax.axis_index("s")
    w = c * NS + s
    r0 = s * RT
    _fill(ones_v, 0.0)
    _zero_acc(ones_v, acc_sh, r0)
    _fill(ones_v, 1.0)
    pltpu.sync_copy(dstb.at[w], idx_d)
    plsc.subcore_barrier()

    for j in range(M):
        pltpu.async_copy(ones_v, acc_sh.at[idx_d.at[j]], dsem, add=True)
    for j in range(M):
        pltpu.make_async_copy(ones_v, acc_sh.at[idx_d.at[j]], dsem).wait()
    plsc.subcore_barrier()
    pltpu.sync_copy(acc_sh.at[pl.ds(r0, RT)], degp.at[c, pl.ds(r0, RT)])


@functools.partial(
    pl.kernel,
    out_type=jax.ShapeDtypeStruct((NC, NACC, D), jnp.float32),
    mesh=_mesh,
    scratch_types=[
        pltpu.VMEM((SG, C), jnp.int32),
        pltpu.VMEM((SG, C), jnp.int32),
        pltpu.VMEM((C, D), jnp.float32),
        pltpu.VMEM((C, D), jnp.float32),
        pltpu.VMEM_SHARED((NACC, D), jnp.float32),
        pltpu.SemaphoreType.DMA,
        pltpu.SemaphoreType.DMA,
    ],
)
def _sc_agg(g, srcb, dstb, accp, idx_s, idx_d, rows0, rows1, acc_sh,
            gs0, gs1):
    c = lax.axis_index("c")
    s = lax.axis_index("s")
    w = c * NS + s
    r0 = s * RT
    rows = (rows0, rows1)
    gsem = (gs0, gs1)
    _fill(rows0, 0.0)
    _zero_acc(rows0, acc_sh, r0)
    plsc.subcore_barrier()

    for h in range(NSTG):
        pltpu.sync_copy(srcb.at[w, h], idx_s)
        pltpu.sync_copy(dstb.at[w, h], idx_d)
        pltpu.async_copy(g.at[idx_s.at[0]], rows[0], gsem[0])
        for j in range(SG):
            b = j % 2
            pltpu.make_async_copy(g.at[idx_s.at[j]], rows[b],
                                  gsem[b]).wait()
            if j + 1 < SG:
                pltpu.async_copy(g.at[idx_s.at[j + 1]], rows[1 - b],
                                 gsem[1 - b])
            pltpu.sync_copy(rows[b], acc_sh.at[idx_d.at[j]], add=True)
    plsc.subcore_barrier()
    pltpu.sync_copy(acc_sh.at[pl.ds(r0, RT)], accp.at[c, pl.ds(r0, RT)])


_R = 1024
_G = -(-NACC // _R)


def _prep_body(x_ref, w_ref, degp_ref, g_ref, dis_ref):
    deg = degp_ref[0, :, 0:1] + degp_ref[1, :, 0:1] + 1.0
    dis = lax.rsqrt(deg)
    h = jnp.dot(x_ref[...], w_ref[...], preferred_element_type=jnp.float32)
    g_ref[...] = h * dis
    dis_ref[...] = dis


@jax.jit
def _tc_prep(x, W1, degp):
    return pl.pallas_call(
        _prep_body,
        grid=(_G,),
        in_specs=[
            pl.BlockSpec((_R, D), lambda i: (i, 0)),
            pl.BlockSpec((D, D), lambda i: (0, 0)),
            pl.BlockSpec((NC, _R, D), lambda i: (0, i, 0)),
        ],
        out_specs=[
            pl.BlockSpec((_R, D), lambda i: (i, 0)),
            pl.BlockSpec((_R, 1), lambda i: (i, 0)),
        ],
        out_shape=[
            jax.ShapeDtypeStruct((N, D), jnp.float32),
            jax.ShapeDtypeStruct((NACC, 1), jnp.float32),
        ],
    )(x, W1, degp)


def _mid_body(accp_ref, g1_ref, dis_ref, b_ref, w_ref, g2_ref):
    a = accp_ref[0] + accp_ref[1] + g1_ref[...]
    z = jnp.maximum(dis_ref[...] * a + b_ref[...], 0.0)
    g2_ref[...] = jnp.dot(z, w_ref[...],
                          preferred_element_type=jnp.float32) * dis_ref[...]


@jax.jit
def _tc_mid(accp, g1, dis, b1, W2):
    return pl.pallas_call(
        _mid_body,
        grid=(_G,),
        in_specs=[
            pl.BlockSpec((NC, _R, D), lambda i: (0, i, 0)),
            pl.BlockSpec((_R, D), lambda i: (i, 0)),
            pl.BlockSpec((_R, 1), lambda i: (i, 0)),
            pl.BlockSpec((1, D), lambda i: (0, 0)),
            pl.BlockSpec((D, D), lambda i: (0, 0)),
        ],
        out_specs=pl.BlockSpec((_R, D), lambda i: (i, 0)),
        out_shape=jax.ShapeDtypeStruct((N, D), jnp.float32),
    )(accp, g1, dis, b1, W2)


def _head_body(accp_ref, g2_ref, dis_ref, b_ref, hw_ref, hb_ref, o_ref):
    a = accp_ref[0] + accp_ref[1] + g2_ref[...]
    z = jnp.maximum(dis_ref[...] * a + b_ref[...], 0.0)
    o_ref[...] = jnp.dot(z, hw_ref[...],
                         preferred_element_type=jnp.float32) + hb_ref[...]


@jax.jit
def _tc_head(accp, g2, dis, b2, head_w, head_b):
    return pl.pallas_call(
        _head_body,
        grid=(_G,),
        in_specs=[
            pl.BlockSpec((NC, _R, D), lambda i: (0, i, 0)),
            pl.BlockSpec((_R, D), lambda i: (i, 0)),
            pl.BlockSpec((_R, 1), lambda i: (i, 0)),
            pl.BlockSpec((1, D), lambda i: (0, 0)),
            pl.BlockSpec((D, 1), lambda i: (0, 0)),
            pl.BlockSpec((1, 1), lambda i: (0, 0)),
        ],
        out_specs=pl.BlockSpec((_R, 1), lambda i: (i, 0)),
        out_shape=jax.ShapeDtypeStruct((N, 1), jnp.float32),
    )(accp, g2, dis, b2, head_w, head_b)


def kernel(x, edge_index, W1, b1, W2, b2, head_w, head_b):
    src, dst = edge_index[0], edge_index[1]
    npad = EPAD - E
    pad_i = jnp.arange(npad, dtype=jnp.int32)
    srcb = jnp.concatenate([src, pad_i % N]).reshape(NW, NSTG, SG, C)
    dstb4 = jnp.concatenate([dst, N + (pad_i % NDUM)]).reshape(
        NW, NSTG, SG, C)
    dstb = dstb4.reshape(NW, M, C)

    degp = _sc_deg(dstb)
    g1, dis = _tc_prep(x, W1, degp)
    accp1 = _sc_agg(g1, srcb, dstb4)
    g2 = _tc_mid(accp1, g1, dis, b1.reshape(1, D), W2)
    accp2 = _sc_agg(g2, srcb, dstb4)
    out = _tc_head(accp2, g2, dis, b2.reshape(1, D), head_w,
                   head_b.reshape(1, 1))
    return out[:, 0]


# speedup vs baseline: 1.1192x; 1.1192x over previous
import functools

import jax
import jax.numpy as jnp
from jax import lax
from jax.experimental import pallas as pl
from jax.experimental.pallas import tpu as pltpu
from jax.experimental.pallas import tpu_sc as plsc

N = 10000
D = 128
E = 320000

NC = 2
NS = 16
NW = NC * NS
C = 128
M = 80
SG = 40
NSTG = M // SG
EPAD = NW * M * C
NACC = 10112
RT = NACC // NS
NDUM = NACC - N

_mesh = plsc.VectorSubcoreMesh(core_axis_name="c", subcore_axis_name="s")

DW = 16


def _fill(ref, val):
    def body(i, carry):
        for t in range(D // 16):
            ref[i, pl.ds(16 * t, 16)] = jnp.full((16,), val, jnp.float32)
        return carry

    lax.fori_loop(0, ref.shape[0], body, 0)


def _zero_acc(zbuf, acc_sh, r0):
    for k in range(4):
        pltpu.sync_copy(zbuf, acc_sh.at[pl.ds(r0 + 128 * k, 128)])
    pltpu.sync_copy(zbuf.at[pl.ds(0, RT - 512)],
                    acc_sh.at[pl.ds(r0 + 512, RT - 512)])


@functools.partial(
    pl.kernel,
    out_type=jax.ShapeDtypeStruct((NC, NACC, D), jnp.float32),
    mesh=_mesh,
    scratch_types=[
        pltpu.VMEM((M, C), jnp.int32),
        pltpu.VMEM((C, D), jnp.float32),
        pltpu.VMEM_SHARED((NACC, D), jnp.float32),
        pltpu.SemaphoreType.DMA,
    ],
)
def _sc_deg(dstb, degp, idx_d, ones_v, acc_sh, dsem):
    c = lax.axis_index("c")
    s = lax.axis_index("s")
    w = c * NS + s
    r0 = s * RT
    _fill(ones_v, 0.0)
    _zero_acc(ones_v, acc_sh, r0)
    _fill(ones_v, 1.0)
    pltpu.sync_copy(dstb.at[w], idx_d)
    plsc.subcore_barrier()

    for j in range(M):
        pltpu.async_copy(ones_v, acc_sh.at[idx_d.at[j]], dsem, add=True)
    for j in range(M):
        pltpu.make_async_copy(ones_v, acc_sh.at[idx_d.at[j]], dsem).wait()
    plsc.subcore_barrier()
    pltpu.sync_copy(acc_sh.at[pl.ds(r0, RT)], degp.at[c, pl.ds(r0, RT)])


@functools.partial(
    pl.kernel,
    out_type=jax.ShapeDtypeStruct((NC, NACC, D), jnp.float32),
    mesh=_mesh,
    scratch_types=[
        pltpu.VMEM((SG, C), jnp.int32),
        pltpu.VMEM((SG, C), jnp.int32),
        pltpu.VMEM((C, D), jnp.float32),
        pltpu.VMEM((C, D), jnp.float32),
        pltpu.VMEM_SHARED((NACC, D), jnp.float32),
        pltpu.SemaphoreType.DMA,
        pltpu.SemaphoreType.DMA,
    ],
)
def _sc_agg(g, srcb, dstb, accp, idx_s, idx_d, rows0, rows1, acc_sh,
            gs0, gs1):
    c = lax.axis_index("c")
    s = lax.axis_index("s")
    w = c * NS + s
    r0 = s * RT
    rows = (rows0, rows1)
    gsem = (gs0, gs1)
    _fill(rows0, 0.0)
    _zero_acc(rows0, acc_sh, r0)
    plsc.subcore_barrier()

    for h in range(NSTG):
        pltpu.sync_copy(srcb.at[w, h], idx_s)
        pltpu.sync_copy(dstb.at[w, h], idx_d)
        pltpu.async_copy(g.at[idx_s.at[0]], rows[0], gsem[0])
        pltpu.async_copy(g.at[idx_s.at[1]], rows[1], gsem[1])
        for j in range(SG):
            b = j % 2
            pltpu.make_async_copy(g.at[idx_s.at[j]], rows[b],
                                  gsem[b]).wait()
            pltpu.sync_copy(rows[b], acc_sh.at[idx_d.at[j]], add=True)
            if j + 2 < SG:
                pltpu.async_copy(g.at[idx_s.at[j + 2]], rows[b], gsem[b])
    plsc.subcore_barrier()
    pltpu.sync_copy(acc_sh.at[pl.ds(r0, RT)], accp.at[c, pl.ds(r0, RT)])


_R = 1024
_G = -(-NACC // _R)


def _prep_body(x_ref, w_ref, degp_ref, g_ref, dis_ref):
    deg = degp_ref[0, :, 0:1] + degp_ref[1, :, 0:1] + 1.0
    dis = lax.rsqrt(deg)
    h = jnp.dot(x_ref[...], w_ref[...], preferred_element_type=jnp.float32)
    g_ref[...] = h * dis
    dis_ref[...] = dis


@jax.jit
def _tc_prep(x, W1, degp):
    return pl.pallas_call(
        _prep_body,
        grid=(_G,),
        in_specs=[
            pl.BlockSpec((_R, D), lambda i: (i, 0)),
            pl.BlockSpec((D, D), lambda i: (0, 0)),
            pl.BlockSpec((NC, _R, D), lambda i: (0, i, 0)),
        ],
        out_specs=[
            pl.BlockSpec((_R, D), lambda i: (i, 0)),
            pl.BlockSpec((_R, 1), lambda i: (i, 0)),
        ],
        out_shape=[
            jax.ShapeDtypeStruct((N, D), jnp.float32),
            jax.ShapeDtypeStruct((NACC, 1), jnp.float32),
        ],
    )(x, W1, degp)


def _mid_body(accp_ref, g1_ref, dis_ref, b_ref, w_ref, g2_ref):
    a = accp_ref[0] + accp_ref[1] + g1_ref[...]
    z = jnp.maximum(dis_ref[...] * a + b_ref[...], 0.0)
    g2_ref[...] = jnp.dot(z, w_ref[...],
                          preferred_element_type=jnp.float32) * dis_ref[...]


@jax.jit
def _tc_mid(accp, g1, dis, b1, W2):
    return pl.pallas_call(
        _mid_body,
        grid=(_G,),
        in_specs=[
            pl.BlockSpec((NC, _R, D), lambda i: (0, i, 0)),
            pl.BlockSpec((_R, D), lambda i: (i, 0)),
            pl.BlockSpec((_R, 1), lambda i: (i, 0)),
            pl.BlockSpec((1, D), lambda i: (0, 0)),
            pl.BlockSpec((D, D), lambda i: (0, 0)),
        ],
        out_specs=pl.BlockSpec((_R, D), lambda i: (i, 0)),
        out_shape=jax.ShapeDtypeStruct((N, D), jnp.float32),
    )(accp, g1, dis, b1, W2)


def _head_body(accp_ref, g2_ref, dis_ref, b_ref, hw_ref, hb_ref, o_ref):
    a = accp_ref[0] + accp_ref[1] + g2_ref[...]
    z = jnp.maximum(dis_ref[...] * a + b_ref[...], 0.0)
    o_ref[...] = jnp.dot(z, hw_ref[...],
                         preferred_element_type=jnp.float32) + hb_ref[...]


@jax.jit
def _tc_head(accp, g2, dis, b2, head_w, head_b):
    return pl.pallas_call(
        _head_body,
        grid=(_G,),
        in_specs=[
            pl.BlockSpec((NC, _R, D), lambda i: (0, i, 0)),
            pl.BlockSpec((_R, D), lambda i: (i, 0)),
            pl.BlockSpec((_R, 1), lambda i: (i, 0)),
            pl.BlockSpec((1, D), lambda i: (0, 0)),
            pl.BlockSpec((D, 1), lambda i: (0, 0)),
            pl.BlockSpec((1, 1), lambda i: (0, 0)),
        ],
        out_specs=pl.BlockSpec((_R, 1), lambda i: (i, 0)),
        out_shape=jax.ShapeDtypeStruct((N, 1), jnp.float32),
    )(accp, g2, dis, b2, head_w, head_b)


def kernel(x, edge_index, W1, b1, W2, b2, head_w, head_b):
    src, dst = edge_index[0], edge_index[1]
    npad = EPAD - E
    pad_i = jnp.arange(npad, dtype=jnp.int32)
    srcb = jnp.concatenate([src, pad_i % N]).reshape(NW, NSTG, SG, C)
    dstb4 = jnp.concatenate([dst, N + (pad_i % NDUM)]).reshape(
        NW, NSTG, SG, C)
    dstb = dstb4.reshape(NW, M, C)

    degp = _sc_deg(dstb)
    g1, dis = _tc_prep(x, W1, degp)
    accp1 = _sc_agg(g1, srcb, dstb4)
    g2 = _tc_mid(accp1, g1, dis, b1.reshape(1, D), W2)
    accp2 = _sc_agg(g2, srcb, dstb4)
    out = _tc_head(accp2, g2, dis, b2.reshape(1, D), head_w,
                   head_b.reshape(1, 1))
    return out[:, 0]


# speedup vs baseline: 1.1198x; 1.0006x over previous
import functools

import jax
import jax.numpy as jnp
from jax import lax
from jax.experimental import pallas as pl
from jax.experimental.pallas import tpu as pltpu
from jax.experimental.pallas import tpu_sc as plsc

N = 10000
D = 128
E = 320000

NC = 2
NS = 16
NW = NC * NS
C = 128
M = 80
SG = 40
NSTG = M // SG
EPAD = NW * M * C
NACC = 10112
RT = NACC // NS
NDUM = NACC - N

_mesh = plsc.VectorSubcoreMesh(core_axis_name="c", subcore_axis_name="s")


def _fill(ref, val):
    def body(i, carry):
        for t in range(D // 16):
            ref[i, pl.ds(16 * t, 16)] = jnp.full((16,), val, jnp.float32)
        return carry

    lax.fori_loop(0, ref.shape[0], body, 0)


def _zero_acc(zbuf, acc_sh, r0):
    for k in range(4):
        pltpu.sync_copy(zbuf, acc_sh.at[pl.ds(r0 + 128 * k, 128)])
    pltpu.sync_copy(zbuf.at[pl.ds(0, RT - 512)],
                    acc_sh.at[pl.ds(r0 + 512, RT - 512)])


@functools.partial(
    pl.kernel,
    out_type=jax.ShapeDtypeStruct((NC, NACC, D), jnp.float32),
    mesh=_mesh,
    scratch_types=[
        pltpu.VMEM((M, C), jnp.int32),
        pltpu.VMEM((C, D), jnp.float32),
        pltpu.VMEM_SHARED((NACC, D), jnp.float32),
        pltpu.SemaphoreType.DMA,
    ],
)
def _sc_deg(dstb, degp, idx_d, ones_v, acc_sh, dsem):
    c = lax.axis_index("c")
    s = lax.axis_index("s")
    w = c * NS + s
    r0 = s * RT
    _fill(ones_v, 0.0)
    _zero_acc(ones_v, acc_sh, r0)
    _fill(ones_v, 1.0)
    pltpu.sync_copy(dstb.at[w], idx_d)
    plsc.subcore_barrier()

    for j in range(M):
        pltpu.async_copy(ones_v, acc_sh.at[idx_d.at[j]], dsem, add=True)
    for j in range(M):
        pltpu.make_async_copy(ones_v, acc_sh.at[idx_d.at[j]], dsem).wait()
    plsc.subcore_barrier()
    pltpu.sync_copy(acc_sh.at[pl.ds(r0, RT)], degp.at[c, pl.ds(r0, RT)])


@functools.partial(
    pl.kernel,
    out_type=jax.ShapeDtypeStruct((NC, NACC, D), jnp.float32),
    mesh=_mesh,
    scratch_types=[
        pltpu.VMEM((SG, C), jnp.int32),
        pltpu.VMEM((SG, C), jnp.int32),
        pltpu.VMEM((C, D), jnp.float32),
        pltpu.VMEM((C, D), jnp.float32),
        pltpu.VMEM_SHARED((NACC, D), jnp.float32),
        pltpu.SemaphoreType.DMA,
        pltpu.SemaphoreType.DMA,
    ],
)
def _sc_agg(g, srcb, dstb, accp, idx_s, idx_d, rows0, rows1, acc_sh,
            gs0, gs1):
    c = lax.axis_index("c")
    s = lax.axis_index("s")
    w = c * NS + s
    r0 = s * RT
    rows = (rows0, rows1)
    gsem = (gs0, gs1)
    _fill(rows0, 0.0)
    _zero_acc(rows0, acc_sh, r0)
    plsc.subcore_barrier()

    for h in range(NSTG):
        pltpu.sync_copy(srcb.at[w, h], idx_s)
        pltpu.sync_copy(dstb.at[w, h], idx_d)
        pltpu.async_copy(g.at[idx_s.at[0]], rows[0], gsem[0])
        pltpu.async_copy(g.at[idx_s.at[1]], rows[1], gsem[1])
        for j in range(SG):
            b = j % 2
            pltpu.make_async_copy(g.at[idx_s.at[j]], rows[b],
                                  gsem[b]).wait()
            pltpu.sync_copy(rows[b], acc_sh.at[idx_d.at[j]], add=True)
            if j + 2 < SG:
                pltpu.async_copy(g.at[idx_s.at[j + 2]], rows[b], gsem[b])
    plsc.subcore_barrier()
    pltpu.sync_copy(acc_sh.at[pl.ds(r0, RT)], accp.at[c, pl.ds(r0, RT)])


_R = 1024
_G = -(-NACC // _R)


def _prep_body(x_ref, w_ref, degp_ref, g_ref, dis_ref):
    deg = degp_ref[0, :, 0:1] + degp_ref[1, :, 0:1] + 1.0
    dis = lax.rsqrt(deg)
    h = jnp.dot(x_ref[...], w_ref[...], preferred_element_type=jnp.float32)
    g_ref[...] = h * dis
    dis_ref[...] = dis


@jax.jit
def _tc_prep(x, W1, degp):
    return pl.pallas_call(
        _prep_body,
        grid=(_G,),
        in_specs=[
            pl.BlockSpec((_R, D), lambda i: (i, 0)),
            pl.BlockSpec((D, D), lambda i: (0, 0)),
            pl.BlockSpec((NC, _R, D), lambda i: (0, i, 0)),
        ],
        out_specs=[
            pl.BlockSpec((_R, D), lambda i: (i, 0)),
            pl.BlockSpec((_R, 1), lambda i: (i, 0)),
        ],
        out_shape=[
            jax.ShapeDtypeStruct((N, D), jnp.float32),
            jax.ShapeDtypeStruct((NACC, 1), jnp.float32),
        ],
    )(x, W1, degp)


def _mid_body(accp_ref, g1_ref, dis_ref, b_ref, w_ref, g2_ref):
    a = accp_ref[0] + accp_ref[1] + g1_ref[...]
    z = jnp.maximum(dis_ref[...] * a + b_ref[...], 0.0)
    g2_ref[...] = jnp.dot(z, w_ref[...],
                          preferred_element_type=jnp.float32) * dis_ref[...]


@jax.jit
def _tc_mid(accp, g1, dis, b1, W2):
    return pl.pallas_call(
        _mid_body,
        grid=(_G,),
        in_specs=[
            pl.BlockSpec((NC, _R, D), lambda i: (0, i, 0)),
            pl.BlockSpec((_R, D), lambda i: (i, 0)),
            pl.BlockSpec((_R, 1), lambda i: (i, 0)),
            pl.BlockSpec((1, D), lambda i: (0, 0)),
            pl.BlockSpec((D, D), lambda i: (0, 0)),
        ],
        out_specs=pl.BlockSpec((_R, D), lambda i: (i, 0)),
        out_shape=jax.ShapeDtypeStruct((N, D), jnp.float32),
    )(accp, g1, dis, b1, W2)


def _head_body(accp_ref, g2_ref, dis_ref, b_ref, hw_ref, hb_ref, o_ref):
    a = accp_ref[0] + accp_ref[1] + g2_ref[...]
    z = jnp.maximum(dis_ref[...] * a + b_ref[...], 0.0)
    o_ref[...] = jnp.dot(z, hw_ref[...],
                         preferred_element_type=jnp.float32) + hb_ref[...]


@jax.jit
def _tc_head(accp, g2, dis, b2, head_w, head_b):
    return pl.pallas_call(
        _head_body,
        grid=(_G,),
        in_specs=[
            pl.BlockSpec((NC, _R, D), lambda i: (0, i, 0)),
            pl.BlockSpec((_R, D), lambda i: (i, 0)),
            pl.BlockSpec((_R, 1), lambda i: (i, 0)),
            pl.BlockSpec((1, D), lambda i: (0, 0)),
            pl.BlockSpec((D, 1), lambda i: (0, 0)),
            pl.BlockSpec((1, 1), lambda i: (0, 0)),
        ],
        out_specs=pl.BlockSpec((_R, 1), lambda i: (i, 0)),
        out_shape=jax.ShapeDtypeStruct((N, 1), jnp.float32),
    )(accp, g2, dis, b2, head_w, head_b)


def kernel(x, edge_index, W1, b1, W2, b2, head_w, head_b):
    src, dst = edge_index[0], edge_index[1]
    npad = EPAD - E
    pad_i = jnp.arange(npad, dtype=jnp.int32)
    srcb = jnp.concatenate([src, pad_i % N]).reshape(NW, NSTG, SG, C)
    dstb4 = jnp.concatenate([dst, N + (pad_i % NDUM)]).reshape(
        NW, NSTG, SG, C)
    dstb = dstb4.reshape(NW, M, C)

    degp = _sc_deg(dstb)
    g1, dis = _tc_prep(x, W1, degp)
    accp1 = _sc_agg(g1, srcb, dstb4)
    g2 = _tc_mid(accp1, g1, dis, b1.reshape(1, D), W2)
    accp2 = _sc_agg(g2, srcb, dstb4)
    out = _tc_head(accp2, g2, dis, b2.reshape(1, D), head_w,
                   head_b.reshape(1, 1))
    return out[:, 0]
